# speedup vs baseline: 2.8389x; 2.8389x over previous
import jax
import jax.numpy as jnp
from jax.experimental import pallas as pl
from jax.experimental.pallas import tpu as pltpu

LANE = 128


def _fwd_body(x_ref, w_ref, b_ref, y_ref):
    in_dim = x_ref.shape[1]
    x = x_ref[...].astype(jnp.bfloat16)

    h = jnp.dot(x, w_ref[0, :in_dim, :], preferred_element_type=jnp.float32)
    h = jnp.maximum(h + b_ref[0:1, :], 0.0).astype(jnp.bfloat16)

    h = jnp.dot(h, w_ref[1], preferred_element_type=jnp.float32)
    h = jnp.maximum(h + b_ref[1:2, :], 0.0).astype(jnp.bfloat16)

    h = jnp.dot(h, w_ref[2], preferred_element_type=jnp.float32)
    h = jnp.maximum(h + b_ref[2:3, :], 0.0).astype(jnp.bfloat16)

    y = jnp.dot(h, w_ref[3], preferred_element_type=jnp.float32) + b_ref[3:4, :]
    y_ref[...] = y[:, :in_dim]


def _forward(x, w_bf16, b_slab, tm):
    B, in_dim = x.shape
    return pl.pallas_call(
        _fwd_body,
        out_shape=jax.ShapeDtypeStruct((B, in_dim), jnp.float32),
        grid=(B // tm,),
        in_specs=[
            pl.BlockSpec((tm, in_dim), lambda i: (i, 0)),
            pl.BlockSpec((4, LANE, LANE), lambda i: (0, 0, 0)),
            pl.BlockSpec((8, LANE), lambda i: (0, 0)),
        ],
        out_specs=pl.BlockSpec((tm, in_dim), lambda i: (i, 0)),
        compiler_params=pltpu.CompilerParams(
            dimension_semantics=("parallel",),
        ),
    )(x, w_bf16, b_slab)


def kernel(x, w_slab, b_slab):
    B = x.shape[0]
    w_bf16 = w_slab.astype(jnp.bfloat16)

    tm = next((t for t in (512, 256, 128) if B % t == 0), None)
    if tm is not None:
        return _forward(x, w_bf16, b_slab, tm)

    tm = LANE if B >= LANE else max(8, ((B + 7) // 8) * 8)
    b_pad = ((B + tm - 1) // tm) * tm
    x_pad = jnp.pad(x, ((0, b_pad - B), (0, 0)))
    return _forward(x_pad, w_bf16, b_slab, tm)[:B]


# speedup vs baseline: 4.7578x; 1.6759x over previous
import jax
import jax.numpy as jnp
from jax.experimental import pallas as pl
from jax.experimental.pallas import tpu as pltpu

LANE = 128


def _fwd_body(x_ref, w_ref, b_ref, y_ref):
    in_dim = x_ref.shape[1]
    x = x_ref[...].astype(jnp.bfloat16)

    h = jnp.dot(x, w_ref[0, :in_dim, :], preferred_element_type=jnp.float32)
    h = jnp.maximum(h + b_ref[0:1, :], 0.0).astype(jnp.bfloat16)

    h = jnp.dot(h, w_ref[1], preferred_element_type=jnp.float32)
    h = jnp.maximum(h + b_ref[1:2, :], 0.0).astype(jnp.bfloat16)

    h = jnp.dot(h, w_ref[2], preferred_element_type=jnp.float32)
    h = jnp.maximum(h + b_ref[2:3, :], 0.0).astype(jnp.bfloat16)

    y = jnp.dot(h, w_ref[3], preferred_element_type=jnp.float32) + b_ref[3:4, :]
    y_ref[...] = y[:, :in_dim]


def _forward(x, w_bf16, b_slab, tm):
    B, in_dim = x.shape
    return pl.pallas_call(
        _fwd_body,
        out_shape=jax.ShapeDtypeStruct((B, in_dim), jnp.float32),
        grid=(B // tm,),
        in_specs=[
            pl.BlockSpec((tm, in_dim), lambda i: (i, 0)),
            pl.BlockSpec((4, LANE, LANE), lambda i: (0, 0, 0)),
            pl.BlockSpec((8, LANE), lambda i: (0, 0)),
        ],
        out_specs=pl.BlockSpec((tm, in_dim), lambda i: (i, 0)),
        compiler_params=pltpu.CompilerParams(
            dimension_semantics=("parallel",),
        ),
    )(x, w_bf16, b_slab)


def kernel(x, w_slab, b_slab):
    B = x.shape[0]
    w_bf16 = w_slab.astype(jnp.bfloat16)

    tm = next((t for t in (2048, 1024, 512, 256, 128) if B % t == 0), None)
    if tm is not None:
        return _forward(x, w_bf16, b_slab, tm)

    tm = LANE if B >= LANE else max(8, ((B + 7) // 8) * 8)
    b_pad = ((B + tm - 1) // tm) * tm
    x_pad = jnp.pad(x, ((0, b_pad - B), (0, 0)))
    return _forward(x_pad, w_bf16, b_slab, tm)[:B]


# speedup vs baseline: 5.6657x; 1.1908x over previous
import jax
import jax.numpy as jnp
from jax.experimental import pallas as pl
from jax.experimental.pallas import tpu as pltpu

LANE = 128


def _fwd_body(x_ref, w_ref, b_ref, y_ref):
    in_dim = x_ref.shape[1]
    x = x_ref[...].astype(jnp.bfloat16)

    h = jnp.dot(x, w_ref[0, :in_dim, :], preferred_element_type=jnp.float32)
    h = jnp.maximum(h + b_ref[0:1, :], 0.0).astype(jnp.bfloat16)

    h = jnp.dot(h, w_ref[1], preferred_element_type=jnp.float32)
    h = jnp.maximum(h + b_ref[1:2, :], 0.0).astype(jnp.bfloat16)

    h = jnp.dot(h, w_ref[2], preferred_element_type=jnp.float32)
    h = jnp.maximum(h + b_ref[2:3, :], 0.0).astype(jnp.bfloat16)

    y = jnp.dot(h, w_ref[3], preferred_element_type=jnp.float32) + b_ref[3:4, :]
    y_ref[...] = y[:, :in_dim]


def _forward(x, w_bf16, b_slab, tm):
    B, in_dim = x.shape
    return pl.pallas_call(
        _fwd_body,
        out_shape=jax.ShapeDtypeStruct((B, in_dim), jnp.float32),
        grid=(B // tm,),
        in_specs=[
            pl.BlockSpec((tm, in_dim), lambda i: (i, 0)),
            pl.BlockSpec((4, LANE, LANE), lambda i: (0, 0, 0)),
            pl.BlockSpec((8, LANE), lambda i: (0, 0)),
        ],
        out_specs=pl.BlockSpec((tm, in_dim), lambda i: (i, 0)),
        compiler_params=pltpu.CompilerParams(
            dimension_semantics=("parallel",),
        ),
    )(x, w_bf16, b_slab)


def kernel(x, w_slab, b_slab):
    B = x.shape[0]
    w_bf16 = w_slab.astype(jnp.bfloat16)

    tm = next((t for t in (8192, 4096, 2048, 1024, 512, 256, 128) if B % t == 0), None)
    if tm is not None:
        return _forward(x, w_bf16, b_slab, tm)

    tm = LANE if B >= LANE else max(8, ((B + 7) // 8) * 8)
    b_pad = ((B + tm - 1) // tm) * tm
    x_pad = jnp.pad(x, ((0, b_pad - B), (0, 0)))
    return _forward(x_pad, w_bf16, b_slab, tm)[:B]


# speedup vs baseline: 5.7830x; 1.0207x over previous
import jax
import jax.numpy as jnp
from jax.experimental import pallas as pl
from jax.experimental.pallas import tpu as pltpu

LANE = 128


def _fwd_body(x_ref, w_ref, b_ref, y_ref):
    in_dim = x_ref.shape[1]
    x = x_ref[...].astype(jnp.bfloat16)

    h = jnp.dot(x, w_ref[0, :in_dim, :], preferred_element_type=jnp.float32)
    h = jnp.maximum(h + b_ref[0:1, :], 0.0).astype(jnp.bfloat16)

    h = jnp.dot(h, w_ref[1], preferred_element_type=jnp.float32)
    h = jnp.maximum(h + b_ref[1:2, :], 0.0).astype(jnp.bfloat16)

    h = jnp.dot(h, w_ref[2], preferred_element_type=jnp.float32)
    h = jnp.maximum(h + b_ref[2:3, :], 0.0).astype(jnp.bfloat16)

    y = jnp.dot(h, w_ref[3], preferred_element_type=jnp.float32) + b_ref[3:4, :]
    y_ref[...] = y[:, :in_dim]


def _forward(x, w_bf16, b_slab, tm):
    B, in_dim = x.shape
    return pl.pallas_call(
        _fwd_body,
        out_shape=jax.ShapeDtypeStruct((B, in_dim), jnp.float32),
        grid=(B // tm,),
        in_specs=[
            pl.BlockSpec((tm, in_dim), lambda i: (i, 0)),
            pl.BlockSpec((4, LANE, LANE), lambda i: (0, 0, 0)),
            pl.BlockSpec((8, LANE), lambda i: (0, 0)),
        ],
        out_specs=pl.BlockSpec((tm, in_dim), lambda i: (i, 0)),
        compiler_params=pltpu.CompilerParams(
            dimension_semantics=("parallel",),
        ),
    )(x, w_bf16, b_slab)


def kernel(x, w_slab, b_slab):
    B = x.shape[0]
    w_bf16 = w_slab.astype(jnp.bfloat16)

    tm = next((t for t in (16384, 8192, 4096, 2048, 1024, 512, 256, 128) if B % t == 0), None)
    if tm is not None:
        return _forward(x, w_bf16, b_slab, tm)

    tm = LANE if B >= LANE else max(8, ((B + 7) // 8) * 8)
    b_pad = ((B + tm - 1) // tm) * tm
    x_pad = jnp.pad(x, ((0, b_pad - B), (0, 0)))
    return _forward(x_pad, w_bf16, b_slab, tm)[:B]
